# baseline (speedup 1.0000x reference)
import jax
import jax.numpy as jnp
from jax import lax
from jax.experimental import pallas as pl
from jax.experimental.pallas import tpu as pltpu
from jax.experimental.pallas import tpu_sc as plsc

NUM_ENTITIES = 1000000
HALF = 32
EMBED = 64
ROWS = 16384
FEATS = 26
NC = 2
NS = 16
NW = NC * NS
BPW = ROWS // NW
CHUNK = 128
NBUF = 4
N_GROUPS = FEATS


def _body(idx_hbm, bra_hbm, ket_hbm, bra_out, ket_out,
          idx_v, bg_v, kg_v, bstg, kstg, gsems, wsems):
    wid = lax.axis_index("s") * NC + lax.axis_index("c")
    b0 = wid * BPW

    zeros16 = jnp.zeros((16,), jnp.float32)
    iota16 = lax.iota(jnp.int32, 16)

    pltpu.sync_copy(idx_hbm.at[:, pl.ds(b0, BPW)], idx_v)

    def zrow(r, c):
        for s in range(NBUF):
            for h in range(8):
                bstg[s, 32 + r, pl.ds(h * 16, 16)] = zeros16
                kstg[s, r, pl.ds(h * 16, 16)] = zeros16
        return c

    lax.fori_loop(0, HALF, zrow, 0)

    def drain(s, f):
        cols = pl.ds(b0 + s * CHUNK, CHUNK)
        pltpu.make_async_copy(
            bstg.at[s], bra_out.at[f, slice(None), cols], wsems.at[s]
        ).wait()
        pltpu.make_async_copy(
            kstg.at[s], ket_out.at[f, slice(None), cols], wsems.at[s]
        ).wait()

    def group(f, c):
        for s in range(NBUF):
            @pl.when(f >= 1)
            def _(s=s):
                drain(s, f - 1)

            idx_ref = idx_v.at[f, pl.ds(s * CHUNK, CHUNK)]
            pltpu.async_copy(bra_hbm.at[idx_ref], bg_v.at[s], gsems.at[s])
            pltpu.async_copy(ket_hbm.at[idx_ref], kg_v.at[s], gsems.at[s])

        for s in range(NBUF):
            idx_ref = idx_v.at[f, pl.ds(s * CHUNK, CHUNK)]
            pltpu.make_async_copy(
                bra_hbm.at[idx_ref], bg_v.at[s], gsems.at[s]).wait()
            pltpu.make_async_copy(
                ket_hbm.at[idx_ref], kg_v.at[s], gsems.at[s]).wait()

            def tloop(lb, c, s=s):
                rows = iota16 + lb * 16
                lanes = pl.ds(lb * 16, 16)
                for e in range(HALF):
                    cols = jnp.full((16,), e, jnp.int32)
                    bstg[s, e, lanes] = plsc.load_gather(
                        bg_v.at[s], [rows, cols])
                    kstg[s, HALF + e, lanes] = plsc.load_gather(
                        kg_v.at[s], [rows, cols])
                return c

            lax.fori_loop(0, CHUNK // 16, tloop, 0)

            cols = pl.ds(b0 + s * CHUNK, CHUNK)
            pltpu.async_copy(bstg.at[s], bra_out.at[f, slice(None), cols],
                             wsems.at[s])
            pltpu.async_copy(kstg.at[s], ket_out.at[f, slice(None), cols],
                             wsems.at[s])
        return c

    lax.fori_loop(0, N_GROUPS, group, 0)

    for s in range(NBUF):
        drain(s, N_GROUPS - 1)


@jax.jit
def _run(idx_t, bra_w, ket_w):
    mesh = plsc.VectorSubcoreMesh(core_axis_name="c", subcore_axis_name="s")
    out = pl.kernel(
        _body,
        out_type=(
            jax.ShapeDtypeStruct((FEATS, EMBED, ROWS), jnp.float32),
            jax.ShapeDtypeStruct((FEATS, EMBED, ROWS), jnp.float32),
        ),
        mesh=mesh,
        compiler_params=pltpu.CompilerParams(use_tc_tiling_on_sc=False,
                                             needs_layout_passes=False),
        scratch_types=[
            pltpu.VMEM((FEATS, BPW), jnp.int32),
            pltpu.VMEM((NBUF, CHUNK, HALF), jnp.float32),
            pltpu.VMEM((NBUF, CHUNK, HALF), jnp.float32),
            pltpu.VMEM((NBUF, EMBED, CHUNK), jnp.float32),
            pltpu.VMEM((NBUF, EMBED, CHUNK), jnp.float32),
            pltpu.SemaphoreType.DMA((NBUF,)),
            pltpu.SemaphoreType.DMA((NBUF,)),
        ],
    )(idx_t, bra_w, ket_w)
    return out


def kernel(index, bra_w, ket_w):
    idx_t = jnp.transpose(index.astype(jnp.int32))
    p_bra, p_ket = _run(idx_t, bra_w, ket_w)
    return (
        jnp.transpose(p_bra, (2, 0, 1)),
        jnp.transpose(p_ket, (2, 0, 1)),
    )


# speedup vs baseline: 1.0021x; 1.0021x over previous
import jax
import jax.numpy as jnp
from jax import lax
from jax.experimental import pallas as pl
from jax.experimental.pallas import tpu as pltpu
from jax.experimental.pallas import tpu_sc as plsc

NUM_ENTITIES = 1000000
HALF = 32
EMBED = 64
ROWS = 16384
FEATS = 26
NC = 2
NS = 16
NW = NC * NS
BPW = ROWS // NW
CHUNK = 128
NBUF = 4
N_GROUPS = FEATS


def _body(idx_hbm, bra_hbm, ket_hbm, bra_out, ket_out,
          idx_v, bg_v, kg_v, bstg, kstg, gsems, wsems):
    wid = lax.axis_index("s") * NC + lax.axis_index("c")
    b0 = wid * BPW

    zeros16 = jnp.zeros((16,), jnp.float32)
    iota16 = lax.iota(jnp.int32, 16)
    rows_c = [iota16 + (k * 16) for k in range(CHUNK // 16)]

    pltpu.sync_copy(idx_hbm.at[:, pl.ds(b0, BPW)], idx_v)

    def zrow(r, c):
        for s in range(NBUF):
            for h in range(8):
                bstg[s, 32 + r, pl.ds(h * 16, 16)] = zeros16
                kstg[s, r, pl.ds(h * 16, 16)] = zeros16
        return c

    lax.fori_loop(0, HALF, zrow, 0)

    def drain(s, f):
        cols = pl.ds(b0 + s * CHUNK, CHUNK)
        pltpu.make_async_copy(
            bstg.at[s], bra_out.at[f, slice(None), cols], wsems.at[s]
        ).wait()
        pltpu.make_async_copy(
            kstg.at[s], ket_out.at[f, slice(None), cols], wsems.at[s]
        ).wait()

    def group(f, c):
        for s in range(NBUF):
            @pl.when(f >= 1)
            def _(s=s):
                drain(s, f - 1)

            idx_ref = idx_v.at[f, pl.ds(s * CHUNK, CHUNK)]
            pltpu.async_copy(bra_hbm.at[idx_ref], bg_v.at[s], gsems.at[s])
            pltpu.async_copy(ket_hbm.at[idx_ref], kg_v.at[s], gsems.at[s])

        for s in range(NBUF):
            idx_ref = idx_v.at[f, pl.ds(s * CHUNK, CHUNK)]
            pltpu.make_async_copy(
                bra_hbm.at[idx_ref], bg_v.at[s], gsems.at[s]).wait()
            pltpu.make_async_copy(
                ket_hbm.at[idx_ref], kg_v.at[s], gsems.at[s]).wait()

            def tloop(e, c, s=s):
                cols = jnp.zeros((16,), jnp.int32) + e
                for k in range(CHUNK // 16):
                    bstg[s, e, pl.ds(k * 16, 16)] = plsc.load_gather(
                        bg_v.at[s], [rows_c[k], cols])
                for k in range(CHUNK // 16):
                    kstg[s, HALF + e, pl.ds(k * 16, 16)] = plsc.load_gather(
                        kg_v.at[s], [rows_c[k], cols])
                return c

            lax.fori_loop(0, HALF, tloop, 0)

            cols = pl.ds(b0 + s * CHUNK, CHUNK)
            pltpu.async_copy(bstg.at[s], bra_out.at[f, slice(None), cols],
                             wsems.at[s])
            pltpu.async_copy(kstg.at[s], ket_out.at[f, slice(None), cols],
                             wsems.at[s])
        return c

    lax.fori_loop(0, N_GROUPS, group, 0)

    for s in range(NBUF):
        drain(s, N_GROUPS - 1)


@jax.jit
def _run(idx_t, bra_w, ket_w):
    mesh = plsc.VectorSubcoreMesh(core_axis_name="c", subcore_axis_name="s")
    out = pl.kernel(
        _body,
        out_type=(
            jax.ShapeDtypeStruct((FEATS, EMBED, ROWS), jnp.float32),
            jax.ShapeDtypeStruct((FEATS, EMBED, ROWS), jnp.float32),
        ),
        mesh=mesh,
        compiler_params=pltpu.CompilerParams(use_tc_tiling_on_sc=False,
                                             needs_layout_passes=False),
        scratch_types=[
            pltpu.VMEM((FEATS, BPW), jnp.int32),
            pltpu.VMEM((NBUF, CHUNK, HALF), jnp.float32),
            pltpu.VMEM((NBUF, CHUNK, HALF), jnp.float32),
            pltpu.VMEM((NBUF, EMBED, CHUNK), jnp.float32),
            pltpu.VMEM((NBUF, EMBED, CHUNK), jnp.float32),
            pltpu.SemaphoreType.DMA((NBUF,)),
            pltpu.SemaphoreType.DMA((NBUF,)),
        ],
    )(idx_t, bra_w, ket_w)
    return out


def kernel(index, bra_w, ket_w):
    idx_t = jnp.transpose(index.astype(jnp.int32))
    p_bra, p_ket = _run(idx_t, bra_w, ket_w)
    return (
        jnp.transpose(p_bra, (2, 0, 1)),
        jnp.transpose(p_ket, (2, 0, 1)),
    )


# speedup vs baseline: 1.2035x; 1.2010x over previous
import jax
import jax.numpy as jnp
from jax import lax
from jax.experimental import pallas as pl
from jax.experimental.pallas import tpu as pltpu
from jax.experimental.pallas import tpu_sc as plsc

NUM_ENTITIES = 1000000
HALF = 32
EMBED = 64
ROWS = 16384
FEATS = 26
NC = 2
NS = 16
NW = NC * NS
BPW = ROWS // NW
CHUNK = 128
NBUF = 4
N_GROUPS = FEATS


def _body(idx_hbm, bra_hbm, ket_hbm, bra_out, ket_out,
          idx_v, bg_v, kg_v, bstg, kstg, gsems, wsems):
    wid = lax.axis_index("s") * NC + lax.axis_index("c")
    b0 = wid * BPW

    zeros16 = jnp.zeros((16,), jnp.float32)
    iota16 = lax.iota(jnp.int32, 16)
    rows_c = [iota16 + (k * 16) for k in range(CHUNK // 16)]

    pltpu.sync_copy(idx_hbm.at[:, pl.ds(b0, BPW)], idx_v)

    def zrow(r, c):
        for s in range(NBUF):
            for h in range(8):
                bstg[s, 32 + r, pl.ds(h * 16, 16)] = zeros16
                kstg[s, r, pl.ds(h * 16, 16)] = zeros16
        return c

    lax.fori_loop(0, HALF, zrow, 0)

    def drain(s, f):
        cols = pl.ds(b0 + s * CHUNK, CHUNK)
        pltpu.make_async_copy(
            bstg.at[s], bra_out.at[f, slice(None), cols], wsems.at[s]
        ).wait()
        pltpu.make_async_copy(
            kstg.at[s], ket_out.at[f, slice(None), cols], wsems.at[s]
        ).wait()

    def group(f, c):
        for s in range(NBUF):
            @pl.when(f >= 1)
            def _(s=s):
                drain(s, f - 1)

            idx_ref = idx_v.at[f, pl.ds(s * CHUNK, CHUNK)]
            pltpu.async_copy(bra_hbm.at[idx_ref], bg_v.at[s], gsems.at[s])
            pltpu.async_copy(ket_hbm.at[idx_ref], kg_v.at[s], gsems.at[s])

        for s in range(NBUF):
            idx_ref = idx_v.at[f, pl.ds(s * CHUNK, CHUNK)]
            pltpu.make_async_copy(
                bra_hbm.at[idx_ref], bg_v.at[s], gsems.at[s]).wait()
            pltpu.make_async_copy(
                ket_hbm.at[idx_ref], kg_v.at[s], gsems.at[s]).wait()

            @plsc.parallel_loop(0, HALF, unroll=2)
            def tloop(e, s=s):
                cols = jnp.zeros((16,), jnp.int32) + e
                for k in range(CHUNK // 16):
                    bstg[s, e, pl.ds(k * 16, 16)] = plsc.load_gather(
                        bg_v.at[s], [rows_c[k], cols])
                for k in range(CHUNK // 16):
                    kstg[s, HALF + e, pl.ds(k * 16, 16)] = plsc.load_gather(
                        kg_v.at[s], [rows_c[k], cols])

            cols = pl.ds(b0 + s * CHUNK, CHUNK)
            pltpu.async_copy(bstg.at[s], bra_out.at[f, slice(None), cols],
                             wsems.at[s])
            pltpu.async_copy(kstg.at[s], ket_out.at[f, slice(None), cols],
                             wsems.at[s])
        return c

    lax.fori_loop(0, N_GROUPS, group, 0)

    for s in range(NBUF):
        drain(s, N_GROUPS - 1)


@jax.jit
def _run(idx_t, bra_w, ket_w):
    mesh = plsc.VectorSubcoreMesh(core_axis_name="c", subcore_axis_name="s")
    out = pl.kernel(
        _body,
        out_type=(
            jax.ShapeDtypeStruct((FEATS, EMBED, ROWS), jnp.float32),
            jax.ShapeDtypeStruct((FEATS, EMBED, ROWS), jnp.float32),
        ),
        mesh=mesh,
        compiler_params=pltpu.CompilerParams(use_tc_tiling_on_sc=False,
                                             needs_layout_passes=False),
        scratch_types=[
            pltpu.VMEM((FEATS, BPW), jnp.int32),
            pltpu.VMEM((NBUF, CHUNK, HALF), jnp.float32),
            pltpu.VMEM((NBUF, CHUNK, HALF), jnp.float32),
            pltpu.VMEM((NBUF, EMBED, CHUNK), jnp.float32),
            pltpu.VMEM((NBUF, EMBED, CHUNK), jnp.float32),
            pltpu.SemaphoreType.DMA((NBUF,)),
            pltpu.SemaphoreType.DMA((NBUF,)),
        ],
    )(idx_t, bra_w, ket_w)
    return out


def kernel(index, bra_w, ket_w):
    idx_t = jnp.transpose(index.astype(jnp.int32))
    p_bra, p_ket = _run(idx_t, bra_w, ket_w)
    return (
        jnp.transpose(p_bra, (2, 0, 1)),
        jnp.transpose(p_ket, (2, 0, 1)),
    )


# speedup vs baseline: 1.4151x; 1.1758x over previous
import jax
import jax.numpy as jnp
from jax import lax
from jax.experimental import pallas as pl
from jax.experimental.pallas import tpu as pltpu
from jax.experimental.pallas import tpu_sc as plsc

NUM_ENTITIES = 1000000
HALF = 32
EMBED = 64
ROWS = 16384
FEATS = 26
NC = 2
NS = 16
NW = NC * NS
BPW = ROWS // NW
CHUNK = 128
NBUF = 4
N_GROUPS = FEATS
EB = EMBED // 8
BB = ROWS // CHUNK


def _body(idx_hbm, bra_hbm, ket_hbm, bra_out, ket_out,
          idx_v, bg_v, kg_v, bstg, kstg, gsems, wsems):
    wid = lax.axis_index("s") * NC + lax.axis_index("c")
    b0 = wid * BPW

    zeros16 = jnp.zeros((16,), jnp.float32)
    iota16 = lax.iota(jnp.int32, 16)
    rows_c = [iota16 + (k * 16) for k in range(CHUNK // 16)]

    pltpu.sync_copy(idx_hbm.at[:, pl.ds(b0, BPW)], idx_v)

    def zrow(r, c):
        for s in range(NBUF):
            for h in range(8):
                bstg[s, 4 + r // 8, r % 8, pl.ds(h * 16, 16)] = zeros16
                kstg[s, r // 8, r % 8, pl.ds(h * 16, 16)] = zeros16
        return c

    lax.fori_loop(0, HALF, zrow, 0)

    def drain(s, f):
        bblk = wid * NBUF + s
        pltpu.make_async_copy(
            bstg.at[s], bra_out.at[f, slice(None), bblk], wsems.at[s]
        ).wait()
        pltpu.make_async_copy(
            kstg.at[s], ket_out.at[f, slice(None), bblk], wsems.at[s]
        ).wait()

    def group(f, c):
        for s in range(NBUF):
            @pl.when(f >= 1)
            def _(s=s):
                drain(s, f - 1)

            idx_ref = idx_v.at[f, pl.ds(s * CHUNK, CHUNK)]
            pltpu.async_copy(bra_hbm.at[idx_ref], bg_v.at[s], gsems.at[s])
            pltpu.async_copy(ket_hbm.at[idx_ref], kg_v.at[s], gsems.at[s])

        for s in range(NBUF):
            idx_ref = idx_v.at[f, pl.ds(s * CHUNK, CHUNK)]
            pltpu.make_async_copy(
                bra_hbm.at[idx_ref], bg_v.at[s], gsems.at[s]).wait()
            pltpu.make_async_copy(
                ket_hbm.at[idx_ref], kg_v.at[s], gsems.at[s]).wait()

            @plsc.parallel_loop(0, HALF, unroll=2)
            def tloop(e, s=s):
                cols = jnp.zeros((16,), jnp.int32) + e
                eb = e // 8
                ei = e % 8
                for k in range(CHUNK // 16):
                    bstg[s, eb, ei, pl.ds(k * 16, 16)] = plsc.load_gather(
                        bg_v.at[s], [rows_c[k], cols])
                for k in range(CHUNK // 16):
                    kstg[s, 4 + eb, ei, pl.ds(k * 16, 16)] = plsc.load_gather(
                        kg_v.at[s], [rows_c[k], cols])

            bblk = wid * NBUF + s
            pltpu.async_copy(bstg.at[s], bra_out.at[f, slice(None), bblk],
                             wsems.at[s])
            pltpu.async_copy(kstg.at[s], ket_out.at[f, slice(None), bblk],
                             wsems.at[s])
        return c

    lax.fori_loop(0, N_GROUPS, group, 0)

    for s in range(NBUF):
        drain(s, N_GROUPS - 1)


@jax.jit
def _run(idx_t, bra_w, ket_w):
    mesh = plsc.VectorSubcoreMesh(core_axis_name="c", subcore_axis_name="s")
    out = pl.kernel(
        _body,
        out_type=(
            jax.ShapeDtypeStruct((FEATS, EB, BB, 8, CHUNK), jnp.float32),
            jax.ShapeDtypeStruct((FEATS, EB, BB, 8, CHUNK), jnp.float32),
        ),
        mesh=mesh,
        compiler_params=pltpu.CompilerParams(use_tc_tiling_on_sc=False,
                                             needs_layout_passes=False),
        scratch_types=[
            pltpu.VMEM((FEATS, BPW), jnp.int32),
            pltpu.VMEM((NBUF, CHUNK, HALF), jnp.float32),
            pltpu.VMEM((NBUF, CHUNK, HALF), jnp.float32),
            pltpu.VMEM((NBUF, EB, 8, CHUNK), jnp.float32),
            pltpu.VMEM((NBUF, EB, 8, CHUNK), jnp.float32),
            pltpu.SemaphoreType.DMA((NBUF,)),
            pltpu.SemaphoreType.DMA((NBUF,)),
        ],
    )(idx_t, bra_w, ket_w)
    return out


def _detile(p):
    return p.transpose(2, 4, 0, 1, 3).reshape(ROWS, FEATS, EMBED)


def kernel(index, bra_w, ket_w):
    idx_t = jnp.transpose(index.astype(jnp.int32))
    p_bra, p_ket = _run(idx_t, bra_w, ket_w)
    return (_detile(p_bra), _detile(p_ket))
